# baseline (speedup 1.0000x reference)
import jax
import jax.numpy as jnp
from jax.experimental import pallas as pl

_P, _N, _NFEAT, _NHID, _SHID = 3, 4096, 128, 64, 32
_BM = 256


def _fts_body(x_ref, w_ref, f1_ref, f2_ref):
    y = jnp.dot(x_ref[...], w_ref[0], preferred_element_type=jnp.float32)
    f1_ref[0] = y[:_N]
    f2_ref[0] = y[_N:]


def _gcn_body(adj_ref, f1_ref, f2_ref, b_ref, a_ref, h1_ref, h2_ref):
    adj = adj_ref[0, 0]
    b = b_ref[0]
    a = a_ref[0]
    y1 = jnp.dot(adj, f1_ref[0], preferred_element_type=jnp.float32) + b
    y2 = jnp.dot(adj, f2_ref[0], preferred_element_type=jnp.float32) + b
    h1_ref[0] = jnp.where(y1 >= 0, y1, a * y1)
    h2_ref[0] = jnp.where(y2 >= 0, y2, a * y2)


def _tail_body(h1_ref, h2_ref, msk_ref, sb1_ref, sb2_ref, l1_ref, l2_ref,
               wa_ref, ba_ref, qa_ref, wdt_ref, bd_ref, out_ref):
    wa = wa_ref[...]
    ba = ba_ref[...]
    qa = qa_ref[...]
    h1s, h2s, w1s, w2s = [], [], [], []
    for i in range(_P):
        h1 = h1_ref[i]
        h2 = h2_ref[i]
        h1s.append(h1)
        h2s.append(h2)
        t1 = jnp.tanh(jnp.dot(h1, wa, preferred_element_type=jnp.float32) + ba)
        t2 = jnp.tanh(jnp.dot(h2, wa, preferred_element_type=jnp.float32) + ba)
        w1s.append(jnp.sum(t1 * qa) / _N)
        w2s.append(jnp.sum(t2 * qa) / _N)

    def _softmax3(ws):
        m = jnp.maximum(jnp.maximum(ws[0], ws[1]), ws[2])
        es = [jnp.exp(w - m) for w in ws]
        s = es[0] + es[1] + es[2]
        return [e / s for e in es]

    b1 = _softmax3(w1s)
    b2 = _softmax3(w2s)
    ha1 = b1[0] * h1s[0] + b1[1] * h1s[1] + b1[2] * h1s[2]
    ha2 = b2[0] * h2s[0] + b2[1] * h2s[1] + b2[2] * h2s[2]

    msk = msk_ref[...]
    c = jax.nn.sigmoid(jnp.sum(ha1 * msk, axis=0, keepdims=True)
                       / jnp.sum(msk))
    u = jnp.dot(c, wdt_ref[...], preferred_element_type=jnp.float32)
    bd = bd_ref[0, 0]
    sc1 = jnp.sum(ha1 * u, axis=1, keepdims=True) + bd + sb1_ref[...]
    sc2 = jnp.sum(ha2 * u, axis=1, keepdims=True) + bd + sb2_ref[...]

    def _bce(x, t):
        return jnp.maximum(x, 0.0) - x * t + jnp.log1p(jnp.exp(-jnp.abs(x)))

    loss = (jnp.sum(_bce(sc1, l1_ref[...]), keepdims=True)
            + jnp.sum(_bce(sc2, l2_ref[...]), keepdims=True))
    out_ref[...] = loss / (2 * _N)


def kernel(seq1, seq2, lbl, adjs, sparse, msk, samp_bias1, samp_bias2,
           W_gcn, b_gcn, a_prelu, W_att, b_att, q_att, W_disc, b_disc):
    del sparse
    x = jnp.concatenate([seq1[0], seq2[0]], axis=0)

    f1, f2 = pl.pallas_call(
        _fts_body,
        grid=(_P,),
        in_specs=[
            pl.BlockSpec((2 * _N, _NFEAT), lambda i: (0, 0)),
            pl.BlockSpec((1, _NFEAT, _NHID), lambda i: (i, 0, 0)),
        ],
        out_specs=[
            pl.BlockSpec((1, _N, _NHID), lambda i: (i, 0, 0)),
            pl.BlockSpec((1, _N, _NHID), lambda i: (i, 0, 0)),
        ],
        out_shape=[
            jax.ShapeDtypeStruct((_P, _N, _NHID), jnp.float32),
            jax.ShapeDtypeStruct((_P, _N, _NHID), jnp.float32),
        ],
    )(x, W_gcn)

    b3 = b_gcn.reshape(_P, 1, _NHID)
    a3 = jnp.broadcast_to(a_prelu[:, None, None], (_P, 1, _NHID))
    nm = _N // _BM
    hh1, hh2 = pl.pallas_call(
        _gcn_body,
        grid=(_P, nm),
        in_specs=[
            pl.BlockSpec((1, 1, _BM, _N), lambda i, m: (i, 0, m, 0)),
            pl.BlockSpec((1, _N, _NHID), lambda i, m: (i, 0, 0)),
            pl.BlockSpec((1, _N, _NHID), lambda i, m: (i, 0, 0)),
            pl.BlockSpec((1, 1, _NHID), lambda i, m: (i, 0, 0)),
            pl.BlockSpec((1, 1, _NHID), lambda i, m: (i, 0, 0)),
        ],
        out_specs=[
            pl.BlockSpec((1, _BM, _NHID), lambda i, m: (i, m, 0)),
            pl.BlockSpec((1, _BM, _NHID), lambda i, m: (i, m, 0)),
        ],
        out_shape=[
            jax.ShapeDtypeStruct((_P, _N, _NHID), jnp.float32),
            jax.ShapeDtypeStruct((_P, _N, _NHID), jnp.float32),
        ],
    )(adjs, f1, f2, b3, a3)

    loss = pl.pallas_call(
        _tail_body,
        out_shape=jax.ShapeDtypeStruct((1, 1), jnp.float32),
    )(hh1, hh2,
      msk.reshape(_N, 1),
      samp_bias1.reshape(_N, 1), samp_bias2.reshape(_N, 1),
      lbl[:, :_N].reshape(_N, 1), lbl[:, _N:].reshape(_N, 1),
      W_att, b_att.reshape(1, _SHID), q_att.reshape(1, _SHID),
      W_disc.T, b_disc.reshape(1, 1))

    return (loss[0, 0], hh1)


# speedup vs baseline: 1.2946x; 1.2946x over previous
import jax
import jax.numpy as jnp
from jax.experimental import pallas as pl
from jax.experimental.pallas import tpu as pltpu

_P, _N, _NFEAT, _NHID, _SHID = 3, 4096, 128, 64, 32
_BM = 512
_NM = _N // _BM


def _fused_body(adj_ref, s1_ref, s2_ref, wg_ref, b_ref, a_ref,
                msk_ref, sb1_ref, sb2_ref, l1_ref, l2_ref,
                wa_ref, ba_ref, qa_ref, wdt_ref, bd_ref,
                hh1_ref, loss_ref,
                f1_s, f2_s, h1_s, h2_s, t1_s, t2_s):
    i = pl.program_id(0)
    m = pl.program_id(1)

    @pl.when((i == 0) & (m == 0))
    def _init():
        for j in range(_P):
            wj = wg_ref[j]
            f1_s[j] = jnp.dot(s1_ref[0], wj, preferred_element_type=jnp.float32)
            f2_s[j] = jnp.dot(s2_ref[0], wj, preferred_element_type=jnp.float32)
        t1_s[...] = jnp.zeros_like(t1_s)
        t2_s[...] = jnp.zeros_like(t2_s)

    adj = adj_ref[0, 0]
    b = b_ref[0]
    a = a_ref[0]
    wa = wa_ref[...]
    ba = ba_ref[...]
    y1 = jnp.dot(adj, f1_s[i], preferred_element_type=jnp.float32) + b
    y2 = jnp.dot(adj, f2_s[i], preferred_element_type=jnp.float32) + b
    h1 = jnp.where(y1 >= 0, y1, a * y1)
    h2 = jnp.where(y2 >= 0, y2, a * y2)
    hh1_ref[0] = h1
    h1_s[i, pl.ds(m * _BM, _BM), :] = h1
    h2_s[i, pl.ds(m * _BM, _BM), :] = h2
    u1 = jnp.tanh(jnp.dot(h1, wa, preferred_element_type=jnp.float32) + ba)
    u2 = jnp.tanh(jnp.dot(h2, wa, preferred_element_type=jnp.float32) + ba)
    t1_s[i] += jnp.sum(u1, axis=0, keepdims=True)
    t2_s[i] += jnp.sum(u2, axis=0, keepdims=True)

    @pl.when((i == _P - 1) & (m == _NM - 1))
    def _tail():
        qa = qa_ref[...]
        w1s = [jnp.sum(t1_s[j] * qa) / _N for j in range(_P)]
        w2s = [jnp.sum(t2_s[j] * qa) / _N for j in range(_P)]

        def _softmax3(ws):
            mx = jnp.maximum(jnp.maximum(ws[0], ws[1]), ws[2])
            es = [jnp.exp(w - mx) for w in ws]
            s = es[0] + es[1] + es[2]
            return [e / s for e in es]

        b1 = _softmax3(w1s)
        b2 = _softmax3(w2s)
        ha1 = b1[0] * h1_s[0] + b1[1] * h1_s[1] + b1[2] * h1_s[2]
        ha2 = b2[0] * h2_s[0] + b2[1] * h2_s[1] + b2[2] * h2_s[2]

        msk = msk_ref[...]
        r = jnp.dot(msk, ha1, preferred_element_type=jnp.float32)
        c = jax.nn.sigmoid(r / jnp.sum(msk))
        u = jnp.dot(c, wdt_ref[...], preferred_element_type=jnp.float32)
        bd = bd_ref[0, 0]
        sc1 = jnp.sum(ha1 * u, axis=1, keepdims=True) + bd + sb1_ref[...]
        sc2 = jnp.sum(ha2 * u, axis=1, keepdims=True) + bd + sb2_ref[...]

        def _bce(x, t):
            return jnp.maximum(x, 0.0) - x * t + jnp.log1p(jnp.exp(-jnp.abs(x)))

        loss = (jnp.sum(_bce(sc1, l1_ref[...]), keepdims=True)
                + jnp.sum(_bce(sc2, l2_ref[...]), keepdims=True))
        loss_ref[...] = loss / (2 * _N)


def kernel(seq1, seq2, lbl, adjs, sparse, msk, samp_bias1, samp_bias2,
           W_gcn, b_gcn, a_prelu, W_att, b_att, q_att, W_disc, b_disc):
    del sparse
    b3 = b_gcn.reshape(_P, 1, _NHID)
    a3 = jnp.broadcast_to(a_prelu[:, None, None], (_P, 1, _NHID))
    const = lambda i, m: (0, 0)
    const3 = lambda i, m: (0, 0, 0)
    per_i = lambda i, m: (i, 0, 0)
    hh1, loss = pl.pallas_call(
        _fused_body,
        grid=(_P, _NM),
        in_specs=[
            pl.BlockSpec((1, 1, _BM, _N), lambda i, m: (i, 0, m, 0)),
            pl.BlockSpec((1, _N, _NFEAT), const3),
            pl.BlockSpec((1, _N, _NFEAT), const3),
            pl.BlockSpec((_P, _NFEAT, _NHID), const3),
            pl.BlockSpec((1, 1, _NHID), per_i),
            pl.BlockSpec((1, 1, _NHID), per_i),
            pl.BlockSpec((1, _N), const),
            pl.BlockSpec((_N, 1), const),
            pl.BlockSpec((_N, 1), const),
            pl.BlockSpec((_N, 1), const),
            pl.BlockSpec((_N, 1), const),
            pl.BlockSpec((_NHID, _SHID), const),
            pl.BlockSpec((1, _SHID), const),
            pl.BlockSpec((1, _SHID), const),
            pl.BlockSpec((_NHID, _NHID), const),
            pl.BlockSpec((1, 1), const),
        ],
        out_specs=[
            pl.BlockSpec((1, _BM, _NHID), lambda i, m: (i, m, 0)),
            pl.BlockSpec((1, 1), const),
        ],
        out_shape=[
            jax.ShapeDtypeStruct((_P, _N, _NHID), jnp.float32),
            jax.ShapeDtypeStruct((1, 1), jnp.float32),
        ],
        scratch_shapes=[
            pltpu.VMEM((_P, _N, _NHID), jnp.float32),
            pltpu.VMEM((_P, _N, _NHID), jnp.float32),
            pltpu.VMEM((_P, _N, _NHID), jnp.float32),
            pltpu.VMEM((_P, _N, _NHID), jnp.float32),
            pltpu.VMEM((_P, 1, _SHID), jnp.float32),
            pltpu.VMEM((_P, 1, _SHID), jnp.float32),
        ],
    )(adjs, seq1, seq2, W_gcn, b3, a3,
      msk,
      samp_bias1.reshape(_N, 1), samp_bias2.reshape(_N, 1),
      lbl[:, :_N].reshape(_N, 1), lbl[:, _N:].reshape(_N, 1),
      W_att, b_att.reshape(1, _SHID), q_att.reshape(1, _SHID),
      W_disc.T, b_disc.reshape(1, 1))

    return (loss[0, 0], hh1)


# speedup vs baseline: 1.3043x; 1.0075x over previous
import jax
import jax.numpy as jnp
from jax.experimental import pallas as pl
from jax.experimental.pallas import tpu as pltpu

_P, _N, _NFEAT, _NHID, _SHID = 3, 4096, 128, 64, 32
_BM = 512
_NM = _N // _BM


def _fused_body(adjl_ref, adjr_ref, s1_ref, s2_ref, wg_ref, b_ref, a_ref,
                msk_ref, sb1_ref, sb2_ref, l1_ref, l2_ref,
                wa_ref, ba_ref, qa_ref, wdt_ref, bd_ref,
                hh1_ref, loss_ref,
                f1_s, f2_s, h1_s, h2_s, t1_s, t2_s):
    i = pl.program_id(0)
    m = pl.program_id(1)

    @pl.when((i == 0) & (m == 0))
    def _init():
        for j in range(_P):
            wj = wg_ref[j]
            f1_s[j] = jnp.dot(s1_ref[0], wj, preferred_element_type=jnp.float32)
            f2_s[j] = jnp.dot(s2_ref[0], wj, preferred_element_type=jnp.float32)
        t1_s[...] = jnp.zeros_like(t1_s)
        t2_s[...] = jnp.zeros_like(t2_s)

    adjl = adjl_ref[0, 0]
    adjr = adjr_ref[0, 0]
    b = b_ref[0]
    a = a_ref[0]
    wa = wa_ref[...]
    ba = ba_ref[...]
    nh = _N // 2
    f1t = f1_s[i, :nh, :]
    f1b = f1_s[i, nh:, :]
    f2t = f2_s[i, :nh, :]
    f2b = f2_s[i, nh:, :]
    y1 = (jnp.dot(adjl, f1t, preferred_element_type=jnp.float32)
          + jnp.dot(adjr, f1b, preferred_element_type=jnp.float32) + b)
    y2 = (jnp.dot(adjl, f2t, preferred_element_type=jnp.float32)
          + jnp.dot(adjr, f2b, preferred_element_type=jnp.float32) + b)
    h1 = jnp.where(y1 >= 0, y1, a * y1)
    h2 = jnp.where(y2 >= 0, y2, a * y2)
    hh1_ref[0] = h1
    h1_s[i, pl.ds(m * _BM, _BM), :] = h1
    h2_s[i, pl.ds(m * _BM, _BM), :] = h2
    u1 = jnp.tanh(jnp.dot(h1, wa, preferred_element_type=jnp.float32) + ba)
    u2 = jnp.tanh(jnp.dot(h2, wa, preferred_element_type=jnp.float32) + ba)
    t1_s[i] += jnp.sum(u1, axis=0, keepdims=True)
    t2_s[i] += jnp.sum(u2, axis=0, keepdims=True)

    @pl.when((i == _P - 1) & (m == _NM - 1))
    def _tail():
        qa = qa_ref[...]
        w1s = [jnp.sum(t1_s[j] * qa) / _N for j in range(_P)]
        w2s = [jnp.sum(t2_s[j] * qa) / _N for j in range(_P)]

        def _softmax3(ws):
            mx = jnp.maximum(jnp.maximum(ws[0], ws[1]), ws[2])
            es = [jnp.exp(w - mx) for w in ws]
            s = es[0] + es[1] + es[2]
            return [e / s for e in es]

        b1 = _softmax3(w1s)
        b2 = _softmax3(w2s)
        ha1 = b1[0] * h1_s[0] + b1[1] * h1_s[1] + b1[2] * h1_s[2]
        ha2 = b2[0] * h2_s[0] + b2[1] * h2_s[1] + b2[2] * h2_s[2]

        msk = msk_ref[...]
        r = jnp.dot(msk, ha1, preferred_element_type=jnp.float32)
        c = jax.nn.sigmoid(r / jnp.sum(msk))
        u = jnp.dot(c, wdt_ref[...], preferred_element_type=jnp.float32)
        bd = bd_ref[0, 0]
        sc1 = jnp.sum(ha1 * u, axis=1, keepdims=True) + bd + sb1_ref[...]
        sc2 = jnp.sum(ha2 * u, axis=1, keepdims=True) + bd + sb2_ref[...]

        def _bce(x, t):
            return jnp.maximum(x, 0.0) - x * t + jnp.log1p(jnp.exp(-jnp.abs(x)))

        loss = (jnp.sum(_bce(sc1, l1_ref[...]), keepdims=True)
                + jnp.sum(_bce(sc2, l2_ref[...]), keepdims=True))
        loss_ref[...] = loss / (2 * _N)


def kernel(seq1, seq2, lbl, adjs, sparse, msk, samp_bias1, samp_bias2,
           W_gcn, b_gcn, a_prelu, W_att, b_att, q_att, W_disc, b_disc):
    del sparse
    b3 = b_gcn.reshape(_P, 1, _NHID)
    a3 = jnp.broadcast_to(a_prelu[:, None, None], (_P, 1, _NHID))
    const = lambda i, m: (0, 0)
    const3 = lambda i, m: (0, 0, 0)
    per_i = lambda i, m: (i, 0, 0)
    hh1, loss = pl.pallas_call(
        _fused_body,
        grid=(_P, _NM),
        in_specs=[
            pl.BlockSpec((1, 1, _BM, _N // 2), lambda i, m: (i, 0, m, 0)),
            pl.BlockSpec((1, 1, _BM, _N // 2), lambda i, m: (i, 0, m, 1)),
            pl.BlockSpec((1, _N, _NFEAT), const3),
            pl.BlockSpec((1, _N, _NFEAT), const3),
            pl.BlockSpec((_P, _NFEAT, _NHID), const3),
            pl.BlockSpec((1, 1, _NHID), per_i),
            pl.BlockSpec((1, 1, _NHID), per_i),
            pl.BlockSpec((1, _N), const),
            pl.BlockSpec((_N, 1), const),
            pl.BlockSpec((_N, 1), const),
            pl.BlockSpec((_N, 1), const),
            pl.BlockSpec((_N, 1), const),
            pl.BlockSpec((_NHID, _SHID), const),
            pl.BlockSpec((1, _SHID), const),
            pl.BlockSpec((1, _SHID), const),
            pl.BlockSpec((_NHID, _NHID), const),
            pl.BlockSpec((1, 1), const),
        ],
        out_specs=[
            pl.BlockSpec((1, _BM, _NHID), lambda i, m: (i, m, 0)),
            pl.BlockSpec((1, 1), const),
        ],
        out_shape=[
            jax.ShapeDtypeStruct((_P, _N, _NHID), jnp.float32),
            jax.ShapeDtypeStruct((1, 1), jnp.float32),
        ],
        scratch_shapes=[
            pltpu.VMEM((_P, _N, _NHID), jnp.float32),
            pltpu.VMEM((_P, _N, _NHID), jnp.float32),
            pltpu.VMEM((_P, _N, _NHID), jnp.float32),
            pltpu.VMEM((_P, _N, _NHID), jnp.float32),
            pltpu.VMEM((_P, 1, _SHID), jnp.float32),
            pltpu.VMEM((_P, 1, _SHID), jnp.float32),
        ],
    )(adjs, adjs, seq1, seq2, W_gcn, b3, a3,
      msk,
      samp_bias1.reshape(_N, 1), samp_bias2.reshape(_N, 1),
      lbl[:, :_N].reshape(_N, 1), lbl[:, _N:].reshape(_N, 1),
      W_att, b_att.reshape(1, _SHID), q_att.reshape(1, _SHID),
      W_disc.T, b_disc.reshape(1, 1))

    return (loss[0, 0], hh1)
